# Initial kernel scaffold; baseline (speedup 1.0000x reference)
import jax
import jax.numpy as jnp
from jax.experimental import pallas as pl


def kernel(x, emb_0, emb_1, emb_2, emb_3, emb_4, emb_5, emb_6, emb_7, emb_8, W_lin, b_lin):
    raise NotImplementedError("write your pallas kernel here")


# speedup vs baseline: 7.7955x; 7.7955x over previous
import jax
import jax.numpy as jnp
from jax.experimental import pallas as pl

_CAT_DIMS = [119, 9, 12, 12, 10, 6, 6, 2, 2]
_NUM_CAT = len(_CAT_DIMS)
_OFFSETS = [0]
for _d in _CAT_DIMS[:-1]:
    _OFFSETS.append(_OFFSETS[-1] + _d)
_TOTAL = sum(_CAT_DIMS)
_EMB_DIM = 64
_BLOCK = 2000


def _fused_kernel(x_ref, table_ref, w_ref, b_ref, out_ref):
    xb = x_ref[:, :]
    scalar = xb[:, _NUM_CAT:]
    acc = jnp.dot(scalar, w_ref[:, :], preferred_element_type=jnp.float32)
    idx = xb[:, :_NUM_CAT].astype(jnp.int32)
    bsz = xb.shape[0]
    iota = jax.lax.broadcasted_iota(jnp.int32, (bsz, _TOTAL), 1)
    onehot = jnp.zeros((bsz, _TOTAL), jnp.float32)
    for i in range(_NUM_CAT):
        gi = idx[:, i : i + 1] + _OFFSETS[i]
        onehot = onehot + (gi == iota).astype(jnp.float32)
    acc = acc + jnp.dot(onehot, table_ref[:, :], preferred_element_type=jnp.float32)
    out_ref[:, :] = acc + b_ref[:, :]


def kernel(x, emb_0, emb_1, emb_2, emb_3, emb_4, emb_5, emb_6, emb_7, emb_8,
           W_lin, b_lin):
    table = jnp.concatenate(
        [emb_0, emb_1, emb_2, emb_3, emb_4, emb_5, emb_6, emb_7, emb_8], axis=0)
    b2 = b_lin[None, :]
    n, feat = x.shape
    grid = n // _BLOCK
    return pl.pallas_call(
        _fused_kernel,
        grid=(grid,),
        in_specs=[
            pl.BlockSpec((_BLOCK, feat), lambda i: (i, 0)),
            pl.BlockSpec((_TOTAL, _EMB_DIM), lambda i: (0, 0)),
            pl.BlockSpec((W_lin.shape[0], _EMB_DIM), lambda i: (0, 0)),
            pl.BlockSpec((1, _EMB_DIM), lambda i: (0, 0)),
        ],
        out_specs=pl.BlockSpec((_BLOCK, _EMB_DIM), lambda i: (i, 0)),
        out_shape=jax.ShapeDtypeStruct((n, _EMB_DIM), jnp.float32),
    )(x, table, W_lin, b2)


# speedup vs baseline: 14.1424x; 1.8142x over previous
import numpy as np
import jax
import jax.numpy as jnp
from jax.experimental import pallas as pl

_CAT_DIMS = [119, 9, 12, 12, 10, 6, 6, 2, 2]
_NUM_CAT = len(_CAT_DIMS)
_TOTAL = sum(_CAT_DIMS)
_EMB_DIM = 64
_BLOCK = 20000

_route_np = np.zeros((_NUM_CAT, _TOTAL), np.float32)
_local_np = np.zeros((1, _TOTAL), np.float32)
_off = 0
for _i, _d in enumerate(_CAT_DIMS):
    _route_np[_i, _off:_off + _d] = 1.0
    _local_np[0, _off:_off + _d] = np.arange(_d, dtype=np.float32)
    _off += _d


def _fused_kernel(x_ref, table_ref, w_ref, b_ref, route_ref, local_ref, out_ref):
    xb = x_ref[:, :]
    acc = jnp.dot(xb[:, _NUM_CAT:], w_ref[:, :], preferred_element_type=jnp.float32)
    idxf = xb[:, :_NUM_CAT].astype(jnp.int32).astype(jnp.float32)
    pos = jnp.dot(idxf, route_ref[:, :], preferred_element_type=jnp.float32)
    onehot = (pos == local_ref[:, :]).astype(jnp.float32)
    acc = acc + jnp.dot(onehot, table_ref[:, :], preferred_element_type=jnp.float32)
    out_ref[:, :] = acc + b_ref[:, :]


def kernel(x, emb_0, emb_1, emb_2, emb_3, emb_4, emb_5, emb_6, emb_7, emb_8,
           W_lin, b_lin):
    table = jnp.concatenate(
        [emb_0, emb_1, emb_2, emb_3, emb_4, emb_5, emb_6, emb_7, emb_8], axis=0)
    b2 = b_lin[None, :]
    route = jnp.asarray(_route_np)
    local = jnp.asarray(_local_np)
    n, feat = x.shape
    grid = n // _BLOCK
    return pl.pallas_call(
        _fused_kernel,
        grid=(grid,),
        in_specs=[
            pl.BlockSpec((_BLOCK, feat), lambda i: (i, 0)),
            pl.BlockSpec((_TOTAL, _EMB_DIM), lambda i: (0, 0)),
            pl.BlockSpec((W_lin.shape[0], _EMB_DIM), lambda i: (0, 0)),
            pl.BlockSpec((1, _EMB_DIM), lambda i: (0, 0)),
            pl.BlockSpec((_NUM_CAT, _TOTAL), lambda i: (0, 0)),
            pl.BlockSpec((1, _TOTAL), lambda i: (0, 0)),
        ],
        out_specs=pl.BlockSpec((_BLOCK, _EMB_DIM), lambda i: (i, 0)),
        out_shape=jax.ShapeDtypeStruct((n, _EMB_DIM), jnp.float32),
    )(x, table, W_lin, b2, route, local)
